# baseline (speedup 1.0000x reference)
import functools

import jax
import jax.numpy as jnp
from jax.experimental import pallas as pl
from jax.experimental.pallas import tpu as pltpu
from jax.experimental.pallas import tpu_sc as plsc

_N = 1024
_C = 128
_NUM_CLASSES = 100
_K = 20
_LAM = 50.0
_BLK = 128
_NBLK = _N // _BLK
_BIG = 3e38

_NSUB = 16
_NW = 32
_AGG_ROWS = _N // _NSUB
_CF_ROWS = _N // _NW
_CLS_PAD = 128


def _sc_knn_kernel(f_hbm, tgt_hbm, out_hbm,
                   f_v, tgt_v, s_loc, aux_loc, out_v,
                   stripe_v, astripe_v, acc_v, aacc_v,
                   sall_sh, aall_sh):
    sc = jax.lax.axis_index("c")
    sub = jax.lax.axis_index("s")
    wid = sc * _NSUB + sub

    iota = jax.lax.iota(jnp.int32, 16)
    zeros16 = jnp.zeros((16,), jnp.float32)

    def zrow(r, c0):
        for c in range(_C // 16):
            s_loc[r, pl.ds(c * 16, 16)] = zeros16
        aux_loc[r, :] = zeros16
        return c0
    jax.lax.fori_loop(0, _CLS_PAD, zrow, 0)

    agg_base = sub * _AGG_ROWS
    pltpu.sync_copy(tgt_hbm, tgt_v)
    pltpu.sync_copy(f_hbm.at[pl.ds(agg_base, _AGG_ROWS)], f_v)

    def agroup(g, c0):
        tv16 = tgt_v[pl.ds(agg_base + g * 16, 16)]
        for j in range(16):
            t = tv16[j]
            tv = jnp.broadcast_to(t, (16,))
            qacc = zeros16
            for c in range(_C // 16):
                x = f_v[g * 16 + j, pl.ds(c * 16, 16)]
                plsc.addupdate_scatter(s_loc, [tv, iota + c * 16], x)
                qacc = qacc + x * x
            q = jnp.sum(qacc)
            x2 = jnp.where(iota == 0, jnp.float32(1.0), jnp.float32(0.0)) \
                + jnp.where(iota == 1, q, jnp.float32(0.0))
            plsc.addupdate_scatter(aux_loc, [tv, iota], x2)
        return c0
    jax.lax.fori_loop(0, _AGG_ROWS // 16, agroup, 0)

    pltpu.sync_copy(s_loc, sall_sh.at[pl.ds(wid * _CLS_PAD, _CLS_PAD)])
    pltpu.sync_copy(aux_loc, aall_sh.at[pl.ds(wid * _CLS_PAD, _CLS_PAD)])
    plsc.subcore_barrier()

    r0s = sub * 8
    def zacc(r, c0):
        for c in range(_C // 16):
            acc_v[r, pl.ds(c * 16, 16)] = zeros16
        aacc_v[r, :] = zeros16
        return c0
    jax.lax.fori_loop(0, 8, zacc, 0)

    def slot_add(t, c0):
        pltpu.sync_copy(sall_sh.at[pl.ds((sc * _NSUB + t) * _CLS_PAD + r0s, 8)], stripe_v)
        pltpu.sync_copy(aall_sh.at[pl.ds((sc * _NSUB + t) * _CLS_PAD + r0s, 8)], astripe_v)
        def radd(r, c1):
            for c in range(_C // 16):
                acc_v[r, pl.ds(c * 16, 16)] = (
                    acc_v[r, pl.ds(c * 16, 16)]
                    + stripe_v[r, pl.ds(c * 16, 16)])
            aacc_v[r, :] = aacc_v[r, :] + astripe_v[r, :]
            return c1
        return jax.lax.fori_loop(0, 8, radd, c0)
    jax.lax.fori_loop(0, _NSUB, slot_add, 0)

    def cls_row(k, tot):
        g = aacc_v[k, :]
        s_cnt = g[0]
        qcls = g[1]
        ssacc = zeros16
        for c in range(_C // 16):
            srow = acc_v[k, pl.ds(c * 16, 16)]
            ssacc = ssacc + srow * srow
        ss = jnp.sum(ssacc)
        cnt = s_cnt - 1.0
        m = jnp.minimum(cnt, jnp.float32(_K))
        mm = jnp.maximum(m, 1.0)
        inv = jnp.float32(1.0)
        for v in range(2, _K + 1):
            inv = jnp.where(mm == jnp.float32(v), jnp.float32(1.0 / v), inv)
        contrib = m * qcls - (2.0 * inv) * (ss - qcls) \
            + qcls * cnt * (inv * inv)
        return tot + contrib
    total = jax.lax.fori_loop(0, 8, cls_row, jnp.float32(0.0))

    out_v[:] = jnp.where(iota == 0, total, jnp.float32(0.0))
    pltpu.sync_copy(out_v, out_hbm.at[wid])


def _sc_knn_closed(f, t32):
    kfn = pl.kernel(
        _sc_knn_kernel,
        out_type=jax.ShapeDtypeStruct((_NW, 16), jnp.float32),
        mesh=plsc.VectorSubcoreMesh(core_axis_name="c", subcore_axis_name="s"),
        compiler_params=pltpu.CompilerParams(needs_layout_passes=False),
        scratch_types=[
            pltpu.VMEM((_AGG_ROWS, _C), jnp.float32),
            pltpu.VMEM((_N,), jnp.int32),
            pltpu.VMEM((_CLS_PAD, _C), jnp.float32),
            pltpu.VMEM((_CLS_PAD, 16), jnp.float32),
            pltpu.VMEM((16,), jnp.float32),
            pltpu.VMEM((8, _C), jnp.float32),
            pltpu.VMEM((8, 16), jnp.float32),
            pltpu.VMEM((8, _C), jnp.float32),
            pltpu.VMEM((8, 16), jnp.float32),
            pltpu.VMEM_SHARED((2 * _NSUB * _CLS_PAD, _C), jnp.float32),
            pltpu.VMEM_SHARED((2 * _NSUB * _CLS_PAD, 16), jnp.float32),
        ],
    )
    return kfn(f, t32)


def _tc_kernel(f_ref, sc_ref, tcol_ref, trow_ref, out_ref):
    f = f_ref[...]
    tcol = tcol_ref[...]
    trow = trow_ref[...]

    ciota_col = jax.lax.broadcasted_iota(jnp.int32, (_C, 1), 0)
    ciota_row = jax.lax.broadcasted_iota(jnp.int32, (1, _C), 1)

    ohT = (ciota_col == trow).astype(jnp.float32)
    oh = (tcol == ciota_row).astype(jnp.float32)
    q_col = jnp.sum(f * f, axis=1, keepdims=True)
    cc_col = jnp.sum(ohT, axis=1, keepdims=True)
    cnt = jax.lax.dot(oh, cc_col, precision=jax.lax.Precision.HIGHEST) - 1.0

    sc = sc_ref[...]
    smax = jnp.max(sc, axis=1, keepdims=True)
    lse = jnp.log(jnp.sum(jnp.exp(sc - smax), axis=1, keepdims=True)) + smax
    s_t = jnp.sum(jnp.where(tcol == ciota_row, sc, 0.0), axis=1, keepdims=True)
    ce = jnp.sum(lse - s_t) * (1.0 / _N)

    any_big = jnp.any(cnt > jnp.float32(_K))

    def no_correction(_):
        return jnp.float32(0.0)

    def correction(_):
        cs = jax.lax.dot(ohT, f, precision=jax.lax.Precision.HIGHEST)
        cq_col = jax.lax.dot(ohT, q_col, precision=jax.lax.Precision.HIGHEST)
        m_all = jnp.minimum(cnt, jnp.float32(_K))
        mm_all = jnp.maximum(m_all, 1.0)
        s_all = jax.lax.dot(oh, cs, precision=jax.lax.Precision.HIGHEST) - f
        q_sel = jax.lax.dot(oh, cq_col, precision=jax.lax.Precision.HIGHEST) - q_col
        contrib_closed = (
            m_all * q_col
            - (2.0 / mm_all) * jnp.sum(f * s_all, axis=1, keepdims=True)
            + q_sel / (mm_all * mm_all)
        )
        ft = f.T
        cols = jax.lax.broadcasted_iota(jnp.int32, (_BLK, _N), 1)

        acc = jnp.float32(0.0)
        for blk in range(_NBLK):
            r0 = blk * _BLK
            cnt_b = cnt[r0:r0 + _BLK, :]
            has_big = jnp.any(cnt_b > jnp.float32(_K))

            def fix_block(_, r0=r0, cnt_b=cnt_b):
                fi = f[r0:r0 + _BLK, :]
                ti = tcol[r0:r0 + _BLK, :]
                parts = []
                for jc in range(_NBLK):
                    ftc = ft[:, jc * _BLK:(jc + 1) * _BLK]
                    diff = jnp.abs(fi[:, :, None] - ftc[None, :, :] + 1e-6)
                    parts.append(jnp.sum(diff, axis=1))
                d = jnp.concatenate(parts, axis=1)
                rows = r0 + jax.lax.broadcasted_iota(jnp.int32, (_BLK, 1), 0)
                same = (ti == trow) & (rows != cols)
                dm = jnp.where(same, d, jnp.float32(_BIG))
                m_b = jnp.minimum(cnt_b, jnp.float32(_K))
                mm_b = jnp.maximum(m_b, 1.0)

                def step(k, carry):
                    dw, w = carry
                    v = jnp.min(dw, axis=1, keepdims=True)
                    jmin = jnp.min(
                        jnp.where(dw == v, cols, jnp.int32(2**30)),
                        axis=1, keepdims=True,
                    )
                    onehot = cols == jmin
                    sel = k.astype(jnp.float32) < m_b
                    w = w + jnp.where(onehot & sel, 1.0, 0.0)
                    dw = jnp.where(onehot, jnp.float32(_BIG), dw)
                    return dw, w

                _, w = jax.lax.fori_loop(
                    0, _K, step, (dm, jnp.zeros((_BLK, _N), jnp.float32))
                )
                s_g = jax.lax.dot(w, f, precision=jax.lax.Precision.HIGHEST)
                q_g = jax.lax.dot(w, q_col, precision=jax.lax.Precision.HIGHEST)
                qi = q_col[r0:r0 + _BLK, :]
                contrib_g = (
                    m_b * qi
                    - (2.0 / mm_b) * jnp.sum(fi * s_g, axis=1, keepdims=True)
                    + q_g / (mm_b * mm_b)
                )
                contrib_c = contrib_closed[r0:r0 + _BLK, :]
                delta = jnp.where(
                    cnt_b > jnp.float32(_K), contrib_g - contrib_c, 0.0)
                return jnp.sum(delta)

            acc = acc + jax.lax.cond(
                has_big, fix_block, lambda _: jnp.float32(0.0), None)

        return acc

    delta = jax.lax.cond(any_big, correction, no_correction, None)

    out_ref[...] = jnp.full((1, 1), ce + (_LAM * 0.5) * delta,
                            dtype=jnp.float32)


@jax.jit
def kernel(feture, scores, target):
    f = feture.astype(jnp.float32)
    t32 = target.astype(jnp.int32)
    tcol = t32.reshape(_N, 1)
    trow = t32.reshape(1, _N)
    sc_pad = jnp.pad(
        scores.astype(jnp.float32),
        ((0, 0), (0, _C - _NUM_CLASSES)),
        constant_values=-1e30,
    )
    tc_out = pl.pallas_call(
        _tc_kernel,
        out_shape=jax.ShapeDtypeStruct((1, 1), jnp.float32),
    )(f, sc_pad, tcol, trow)
    knn_parts = _sc_knn_closed(f, t32)
    return tc_out[0, 0] + (_LAM * 0.25) * jnp.sum(knn_parts)
